# baseline (speedup 1.0000x reference)
import functools

import jax
import jax.numpy as jnp
from jax import lax
from jax.experimental import pallas as pl
from jax.experimental.pallas import tpu as pltpu
from jax.experimental.pallas import tpu_sc as plsc

_NC, _NS = 2, 16
_NW = _NC * _NS
_D = 64
_CHUNK = 512
_DK = 512


def _tc_depad(table):
    v = table.shape[0]
    blk = 512
    h = ((v + 1) // 2 + blk - 1) // blk * blk
    nb = h // blk

    def body(x1_ref, x2_ref, o_ref):
        o_ref[:, :_D] = x1_ref[...]
        o_ref[:, _D:] = x2_ref[...]

    return pl.pallas_call(
        body,
        grid=(nb,),
        in_specs=[
            pl.BlockSpec((blk, _D), lambda i: (i, 0)),
            pl.BlockSpec((blk, _D), lambda i: (i + nb, 0)),
        ],
        out_specs=pl.BlockSpec((blk, 2 * _D), lambda i: (i, 0)),
        out_shape=jax.ShapeDtypeStruct((h, 2 * _D), jnp.float32),
    )(table, table)


def _sc_gather_packed(table_lin, flat_idx):
    n = flat_idx.shape[0]
    n2 = n // 2
    b_per_w = n // _NW
    n_chunks = b_per_w // _CHUNK
    mesh = plsc.VectorSubcoreMesh(core_axis_name="c", subcore_axis_name="s")

    @functools.partial(
        pl.kernel,
        mesh=mesh,
        compiler_params=pltpu.CompilerParams(use_tc_tiling_on_sc=False),
        out_type=jax.ShapeDtypeStruct((n2, 2 * _D), jnp.float32),
        scratch_types=[
            pltpu.VMEM((_CHUNK,), jnp.int32),
            pltpu.VMEM((_CHUNK, _D), jnp.float32),
            pltpu.SemaphoreType.DMA,
        ],
    )
    def k(table_hbm, idx_hbm, out_hbm, idx_v, rows_v, sem):
        wid = lax.axis_index("s") * _NC + lax.axis_index("c")
        base = wid * b_per_w
        half = wid // (_NW // 2)
        col = half * _D
        rbase = base - half * n2

        def body(c, carry):
            off = base + c * _CHUNK
            roff = rbase + c * _CHUNK
            pltpu.sync_copy(idx_hbm.at[pl.ds(off, _CHUNK)], idx_v)
            pltpu.async_copy(table_hbm.at[idx_v], rows_v, sem).wait()
            pltpu.sync_copy(
                rows_v, out_hbm.at[pl.ds(roff, _CHUNK), pl.ds(col, _D)]
            )
            return carry

        lax.fori_loop(0, n_chunks, body, 0)

    return k(table_lin, flat_idx)


def _tc_project(e2, W, b2):
    n2 = e2.shape[0]
    rows = 2048
    out_dim = W.shape[0]

    def body(e_ref, w_ref, b_ref, o_ref):
        x = e_ref[...]
        g = 0.5 * x * (1.0 + lax.erf(x * 0.7071067811865476))
        w = w_ref[...]
        bb = b_ref[...]
        o_ref[0] = (
            lax.dot_general(
                g[:, :_D], w, (((1,), (1,)), ((), ())),
                preferred_element_type=jnp.float32,
            )
            + bb
        )
        o_ref[1] = (
            lax.dot_general(
                g[:, _D:], w, (((1,), (1,)), ((), ())),
                preferred_element_type=jnp.float32,
            )
            + bb
        )

    return pl.pallas_call(
        body,
        grid=(n2 // rows,),
        in_specs=[
            pl.BlockSpec((rows, 2 * _D), lambda i: (i, 0)),
            pl.BlockSpec((out_dim, _D), lambda i: (0, 0)),
            pl.BlockSpec((1, out_dim), lambda i: (0, 0)),
        ],
        out_specs=pl.BlockSpec((2, rows, out_dim), lambda i: (0, i, 0)),
        out_shape=jax.ShapeDtypeStruct((2, n2, out_dim), jnp.float32),
    )(e2, W, b2)


def kernel(times, table, W, b):
    B, L = times.shape
    t = times.reshape(-1).astype(jnp.int32)
    packed = _tc_depad(table)
    h = packed.shape[0]
    flat_idx = jnp.where(t < h, 2 * t, 2 * (t - h) + 1)
    table_lin = packed.reshape(h * 2, _D)
    e2 = _sc_gather_packed(table_lin, flat_idx)
    out3 = _tc_project(e2, W, b.reshape(1, -1))
    return out3.reshape(B, L, W.shape[0])


# speedup vs baseline: 1.4490x; 1.4490x over previous
import functools

import jax
import jax.numpy as jnp
from jax import lax
from jax.experimental import pallas as pl
from jax.experimental.pallas import tpu as pltpu
from jax.experimental.pallas import tpu_sc as plsc

_NC, _NS = 2, 16
_NW = _NC * _NS
_D = 64
_CHUNK = 512


def _sc_gather_packed(table, flat_idx):
    n = flat_idx.shape[0]
    n2 = n // 2
    b_per_w = n // _NW
    n_chunks = b_per_w // _CHUNK
    mesh = plsc.VectorSubcoreMesh(core_axis_name="c", subcore_axis_name="s")

    @functools.partial(
        pl.kernel,
        mesh=mesh,
        compiler_params=pltpu.CompilerParams(use_tc_tiling_on_sc=False),
        out_type=jax.ShapeDtypeStruct((n2, 2 * _D), jnp.float32),
        scratch_types=[
            pltpu.VMEM((_CHUNK,), jnp.int32),
            pltpu.VMEM((_CHUNK,), jnp.int32),
            pltpu.VMEM((_CHUNK, _D), jnp.float32),
            pltpu.VMEM((_CHUNK, _D), jnp.float32),
            pltpu.SemaphoreType.DMA,
            pltpu.SemaphoreType.DMA,
            pltpu.SemaphoreType.DMA,
            pltpu.SemaphoreType.DMA,
        ],
    )
    def k(table_hbm, idx_hbm, out_hbm, idx_a, idx_b, rows_a, rows_b,
          sga, sgb, soa, sob):
        wid = lax.axis_index("s") * _NC + lax.axis_index("c")
        base = wid * b_per_w
        half = wid // (_NW // 2)
        col = half * _D
        rbase = base - half * n2

        def body(c2, carry):
            ca = 2 * c2
            offa = base + ca * _CHUNK
            roffa = rbase + ca * _CHUNK
            pltpu.sync_copy(idx_hbm.at[pl.ds(offa, _CHUNK)], idx_a)
            ga = pltpu.async_copy(table_hbm.at[idx_a], rows_a, sga)
            pltpu.sync_copy(idx_hbm.at[pl.ds(offa + _CHUNK, _CHUNK)], idx_b)
            gb = pltpu.async_copy(table_hbm.at[idx_b], rows_b, sgb)
            ga.wait()
            sa = pltpu.async_copy(
                rows_a,
                out_hbm.at[pl.ds(roffa, _CHUNK), pl.ds(col, _D)],
                soa,
            )
            gb.wait()
            sb = pltpu.async_copy(
                rows_b,
                out_hbm.at[pl.ds(roffa + _CHUNK, _CHUNK), pl.ds(col, _D)],
                sob,
            )
            sa.wait()
            sb.wait()
            return carry

        lax.fori_loop(0, n_chunks // 2, body, 0)

    return k(table, flat_idx)


def _tc_project(e2, W, b2):
    n2 = e2.shape[0]
    rows = 2048
    out_dim = W.shape[0]

    def body(e_ref, w_ref, b_ref, o_ref):
        x = e_ref[...]
        g = 0.5 * x * (1.0 + lax.erf(x * 0.7071067811865476))
        w = w_ref[...]
        bb = b_ref[...]
        o_ref[0] = (
            lax.dot_general(
                g[:, :_D], w, (((1,), (1,)), ((), ())),
                preferred_element_type=jnp.float32,
            )
            + bb
        )
        o_ref[1] = (
            lax.dot_general(
                g[:, _D:], w, (((1,), (1,)), ((), ())),
                preferred_element_type=jnp.float32,
            )
            + bb
        )

    return pl.pallas_call(
        body,
        grid=(n2 // rows,),
        in_specs=[
            pl.BlockSpec((rows, 2 * _D), lambda i: (i, 0)),
            pl.BlockSpec((out_dim, _D), lambda i: (0, 0)),
            pl.BlockSpec((1, out_dim), lambda i: (0, 0)),
        ],
        out_specs=pl.BlockSpec((2, rows, out_dim), lambda i: (0, i, 0)),
        out_shape=jax.ShapeDtypeStruct((2, n2, out_dim), jnp.float32),
    )(e2, W, b2)


def kernel(times, table, W, b):
    B, L = times.shape
    flat_idx = times.reshape(-1).astype(jnp.int32)
    e2 = _sc_gather_packed(table, flat_idx)
    out3 = _tc_project(e2, W, b.reshape(1, -1))
    return out3.reshape(B, L, W.shape[0])
